# Initial kernel scaffold; baseline (speedup 1.0000x reference)
import jax
import jax.numpy as jnp
from jax.experimental import pallas as pl


def kernel(in_feat, edge_index, W1, b1, W2, b2):
    raise NotImplementedError("write your pallas kernel here")


# speedup vs baseline: 6.3769x; 6.3769x over previous
import functools

import jax
import jax.numpy as jnp
from jax import lax
from jax.experimental import pallas as pl
from jax.experimental.pallas import tpu as pltpu
from jax.experimental.pallas import tpu_sc as plsc

N = 10000
E = 160000
D_IN = 256
D_H = 512
D_OUT = 64

NC, NS = 2, 16
CHUNK = 128
NCH = 1280
EPAD = NCH * CHUNK
CPT = NCH // NS
WCPT = NCH // (NC * NS)
NPAD = 10240
RPT = NPAD // NS
NXP = 10016

@functools.cache
def _mesh():
    return plsc.VectorSubcoreMesh(
        core_axis_name="c", subcore_axis_name="s", num_cores=NC, num_subcores=NS)


def _sc_degrees_body(srcp, dstp, ones_h, z1_h, deg_o, deg_i, idx_v, ones_v, acc):
    cid = lax.axis_index("c")
    sid = lax.axis_index("s")
    pltpu.sync_copy(ones_h, ones_v)
    pltpu.sync_copy(z1_h, acc.at[pl.ds(sid * RPT, RPT)])

    @pl.when(cid == 0)
    def _():
        pltpu.sync_copy(srcp.at[pl.ds(sid * CPT, CPT)], idx_v)

    @pl.when(cid == 1)
    def _():
        pltpu.sync_copy(dstp.at[pl.ds(sid * CPT, CPT)], idx_v)

    plsc.subcore_barrier()

    def body(j, c):
        pltpu.sync_copy(ones_v, acc.at[idx_v.at[j]], add=True)
        return c

    lax.fori_loop(0, CPT, body, 0)
    plsc.subcore_barrier()
    sl = pl.ds(sid * RPT, RPT)

    @pl.when(cid == 0)
    def _():
        pltpu.sync_copy(acc.at[sl], deg_o.at[sl])

    @pl.when(cid == 1)
    def _():
        pltpu.sync_copy(acc.at[sl], deg_i.at[sl])


@functools.cache
def _sc_degrees():
    return pl.kernel(
        _sc_degrees_body,
        out_type=[
            jax.ShapeDtypeStruct((NPAD,), jnp.float32),
            jax.ShapeDtypeStruct((NPAD,), jnp.float32),
        ],
        mesh=_mesh(),
        scratch_types=[
            pltpu.VMEM((CPT, CHUNK), jnp.int32),
            pltpu.VMEM((CHUNK,), jnp.float32),
            pltpu.VMEM_SHARED((NPAD,), jnp.float32),
        ],
    )


def _sc_scatter_body(srcp, dstp, x0, x1, ndp, z2_h, z1_h,
                     agg0, agg1, w0, w1,
                     isrc, idst, iwsrc, iwdst, rows_v, wvals, acc, wacc, sem):
    cid = lax.axis_index("c")
    sid = lax.axis_index("s")
    pltpu.sync_copy(srcp.at[pl.ds(sid * CPT, CPT)], isrc)
    pltpu.sync_copy(dstp.at[pl.ds(sid * CPT, CPT)], idst)
    wbase = (cid * NS + sid) * WCPT
    pltpu.sync_copy(srcp.at[pl.ds(wbase, WCPT)], iwsrc)
    pltpu.sync_copy(dstp.at[pl.ds(wbase, WCPT)], iwdst)
    pltpu.sync_copy(z2_h, acc.at[pl.ds(sid * RPT, RPT)])
    pltpu.sync_copy(z1_h, wacc.at[pl.ds(sid * RPT, RPT)])
    plsc.subcore_barrier()

    def rowloop(xref):
        def body(j, c):
            pltpu.async_copy(xref.at[isrc.at[j]], rows_v, sem).wait()
            pltpu.sync_copy(rows_v, acc.at[idst.at[j]], add=True)
            return c
        lax.fori_loop(0, CPT, body, 0)

    @pl.when(cid == 0)
    def _():
        rowloop(x0)

    @pl.when(cid == 1)
    def _():
        rowloop(x1)

    def wbody(j, c):
        pltpu.async_copy(ndp.at[iwdst.at[j]], wvals, sem).wait()
        pltpu.sync_copy(wvals, wacc.at[iwsrc.at[j]], add=True)
        return c

    lax.fori_loop(0, WCPT, wbody, 0)
    plsc.subcore_barrier()
    sl = pl.ds(sid * RPT, RPT)

    @pl.when(cid == 0)
    def _():
        pltpu.sync_copy(acc.at[sl], agg0.at[sl])
        pltpu.sync_copy(wacc.at[sl], w0.at[sl])

    @pl.when(cid == 1)
    def _():
        pltpu.sync_copy(acc.at[sl], agg1.at[sl])
        pltpu.sync_copy(wacc.at[sl], w1.at[sl])


@functools.cache
def _sc_scatter():
    return pl.kernel(
        _sc_scatter_body,
        out_type=[
            jax.ShapeDtypeStruct((NPAD, 128), jnp.float32),
            jax.ShapeDtypeStruct((NPAD, 128), jnp.float32),
            jax.ShapeDtypeStruct((NPAD,), jnp.float32),
            jax.ShapeDtypeStruct((NPAD,), jnp.float32),
        ],
        mesh=_mesh(),
        scratch_types=[
            pltpu.VMEM((CPT, CHUNK), jnp.int32),
            pltpu.VMEM((CPT, CHUNK), jnp.int32),
            pltpu.VMEM((WCPT, CHUNK), jnp.int32),
            pltpu.VMEM((WCPT, CHUNK), jnp.int32),
            pltpu.VMEM((CHUNK, 128), jnp.float32),
            pltpu.VMEM((CHUNK,), jnp.float32),
            pltpu.VMEM_SHARED((NPAD, 128), jnp.float32),
            pltpu.VMEM_SHARED((NPAD,), jnp.float32),
            pltpu.SemaphoreType.DMA,
        ],
    )


def _tc_prep_body(dof, dif, feat, x_out, ns_out, nd_out):
    do = dof[...]
    di = dif[...]
    ns = jnp.where(do > 0.0, lax.rsqrt(jnp.maximum(do, 1.0)), 0.0)
    nd = jnp.where(di > 0.0, lax.rsqrt(jnp.maximum(di, 1.0)), 0.0)
    ns_out[...] = ns
    nd_out[...] = nd
    x_out[...] = feat[...] * ns


_PREP_BLK = 1000


def _tc_prep(deg_o, deg_i, in_feat):
    grid = N // _PREP_BLK
    return pl.pallas_call(
        _tc_prep_body,
        grid=(grid,),
        in_specs=[
            pl.BlockSpec((_PREP_BLK, 1), lambda i: (i, 0)),
            pl.BlockSpec((_PREP_BLK, 1), lambda i: (i, 0)),
            pl.BlockSpec((_PREP_BLK, D_IN), lambda i: (i, 0)),
        ],
        out_specs=[
            pl.BlockSpec((_PREP_BLK, D_IN), lambda i: (i, 0)),
            pl.BlockSpec((_PREP_BLK, 1), lambda i: (i, 0)),
            pl.BlockSpec((_PREP_BLK, 1), lambda i: (i, 0)),
        ],
        out_shape=[
            jax.ShapeDtypeStruct((N, D_IN), jnp.float32),
            jax.ShapeDtypeStruct((N, 1), jnp.float32),
            jax.ShapeDtypeStruct((N, 1), jnp.float32),
        ],
    )(deg_o, deg_i, in_feat)


def _tc_final_body(a0, a1, ndr, nsr, u0, u1, wa, wb, b1r, w2r, b2r, out, pooled):
    i = pl.program_id(0)
    nd = ndr[...]
    h = jnp.dot(a0[...] * nd, wa[...], preferred_element_type=jnp.float32)
    h = h + jnp.dot(a1[...] * nd, wb[...], preferred_element_type=jnp.float32)
    h = jnp.maximum(h + b1r[...], 0.0)
    coef = (u0[...] + u1[...]) * nsr[...]
    part = jnp.sum(h * coef, axis=0, keepdims=True)

    @pl.when(i == 0)
    def _():
        pooled[...] = part

    @pl.when(i > 0)
    def _():
        pooled[...] += part

    @pl.when(i == pl.num_programs(0) - 1)
    def _():
        out[...] = (jnp.dot(pooled[...] * (1.0 / N), w2r[...],
                            preferred_element_type=jnp.float32) + b2r[...])


def _tc_final(a0, a1, nd2, ns2, u0, u1, wa, wb, b1r, w2r, b2r):
    grid = N // _PREP_BLK
    return pl.pallas_call(
        _tc_final_body,
        grid=(grid,),
        in_specs=[
            pl.BlockSpec((_PREP_BLK, 128), lambda i: (i, 0)),
            pl.BlockSpec((_PREP_BLK, 128), lambda i: (i, 0)),
            pl.BlockSpec((_PREP_BLK, 1), lambda i: (i, 0)),
            pl.BlockSpec((_PREP_BLK, 1), lambda i: (i, 0)),
            pl.BlockSpec((_PREP_BLK, 1), lambda i: (i, 0)),
            pl.BlockSpec((_PREP_BLK, 1), lambda i: (i, 0)),
            pl.BlockSpec((128, D_H), lambda i: (0, 0)),
            pl.BlockSpec((128, D_H), lambda i: (0, 0)),
            pl.BlockSpec((1, D_H), lambda i: (0, 0)),
            pl.BlockSpec((D_H, D_OUT), lambda i: (0, 0)),
            pl.BlockSpec((1, D_OUT), lambda i: (0, 0)),
        ],
        out_specs=pl.BlockSpec((1, D_OUT), lambda i: (0, 0)),
        out_shape=jax.ShapeDtypeStruct((1, D_OUT), jnp.float32),
        scratch_shapes=[pltpu.VMEM((1, D_H), jnp.float32)],
    )(a0, a1, nd2, ns2, u0, u1, wa, wb, b1r, w2r, b2r)


def kernel(in_feat, edge_index, W1, b1, W2, b2):
    src = edge_index[0]
    dst = edge_index[1]
    pad = jnp.full((EPAD - E,), N, jnp.int32)
    srcp = jnp.concatenate([src, pad]).reshape(NCH, CHUNK)
    dstp = jnp.concatenate([dst, pad]).reshape(NCH, CHUNK)
    ones_h = jnp.ones((CHUNK,), jnp.float32)
    z1 = jnp.zeros((RPT,), jnp.float32)
    z2 = jnp.zeros((RPT, 128), jnp.float32)

    deg_o, deg_i = _sc_degrees()(srcp, dstp, ones_h, z1)
    x, ns2, nd2 = _tc_prep(deg_o[:N, None], deg_i[:N, None], in_feat)
    x0 = jnp.pad(x[:, :128], ((0, NXP - N), (0, 0)))
    x1 = jnp.pad(x[:, 128:], ((0, NXP - N), (0, 0)))
    ndp = jnp.pad(nd2[:, 0], (0, NXP - N))
    agg0, agg1, w0, w1 = _sc_scatter()(srcp, dstp, x0, x1, ndp, z2, z1)
    out = _tc_final(agg0[:N], agg1[:N], nd2, ns2,
                    w0[:N, None], w1[:N, None],
                    W1[:128], W1[128:], b1[None], W2, b2[None])
    return out.reshape(D_OUT)


# speedup vs baseline: 7.1112x; 1.1152x over previous
import functools

import jax
import jax.numpy as jnp
from jax import lax
from jax.experimental import pallas as pl
from jax.experimental.pallas import tpu as pltpu
from jax.experimental.pallas import tpu_sc as plsc

N = 10000
E = 160000
D_IN = 256
D_H = 512
D_OUT = 64

NC, NS = 2, 16
CHUNK = 128
NCH = 1280
EPAD = NCH * CHUNK
CPT = NCH // NS
HCPT = CPT // 2
WCPT = NCH // (NC * NS)
NPAD1 = 10240
RPT1 = NPAD1 // NS
NPAD2 = 10112
RPT2 = NPAD2 // NS
NXP = 10016


@functools.cache
def _mesh():
    return plsc.VectorSubcoreMesh(
        core_axis_name="c", subcore_axis_name="s", num_cores=NC, num_subcores=NS)


def _sc_degrees_body(srcp, dstp, ones_h, z1_h, deg_o, deg_i, idx_v, ones_v, acc,
                     dsem):
    cid = lax.axis_index("c")
    sid = lax.axis_index("s")
    d1 = pltpu.async_copy(ones_h, ones_v, dsem)
    d2 = pltpu.async_copy(z1_h, acc.at[pl.ds(sid * RPT1, RPT1)], dsem)

    @pl.when(cid == 0)
    def _():
        pltpu.sync_copy(srcp.at[pl.ds(sid * CPT, CPT)], idx_v)

    @pl.when(cid == 1)
    def _():
        pltpu.sync_copy(dstp.at[pl.ds(sid * CPT, CPT)], idx_v)

    d1.wait()
    d2.wait()
    plsc.subcore_barrier()

    def fire(j, c):
        pltpu.async_copy(ones_v, acc.at[idx_v.at[j]], dsem, add=True)
        return c

    lax.fori_loop(0, CPT, fire, 0)

    def drain(j, c):
        pltpu.make_async_copy(ones_v, acc.at[idx_v.at[0]], dsem).wait()
        return c

    lax.fori_loop(0, CPT, drain, 0)
    plsc.subcore_barrier()
    sl = pl.ds(sid * RPT1, RPT1)

    @pl.when(cid == 0)
    def _():
        pltpu.sync_copy(acc.at[sl], deg_o.at[sl])

    @pl.when(cid == 1)
    def _():
        pltpu.sync_copy(acc.at[sl], deg_i.at[sl])


@functools.cache
def _sc_degrees():
    return pl.kernel(
        _sc_degrees_body,
        out_type=[
            jax.ShapeDtypeStruct((NPAD1,), jnp.float32),
            jax.ShapeDtypeStruct((NPAD1,), jnp.float32),
        ],
        mesh=_mesh(),
        scratch_types=[
            pltpu.VMEM((CPT, CHUNK), jnp.int32),
            pltpu.VMEM((CHUNK,), jnp.float32),
            pltpu.VMEM_SHARED((NPAD1,), jnp.float32),
            pltpu.SemaphoreType.DMA,
        ],
    )


def _sc_scatter_body(srcp, dstp, x0, x1, ndp, z2_h, z1_h,
                     agg0, agg1, w0, w1,
                     isrc, idst, rows_a, rows_b, wvals, acc, wacc,
                     gsem, sasem, sbsem, wsem):
    cid = lax.axis_index("c")
    sid = lax.axis_index("s")
    sl2 = pl.ds(sid * RPT2, RPT2)
    sl1 = pl.ds(sid * RPT1, RPT1)
    pltpu.sync_copy(z2_h, acc.at[sl2])
    pltpu.sync_copy(z1_h, wacc.at[sl1])
    plsc.subcore_barrier()

    def load_half(h):
        d1 = pltpu.async_copy(
            srcp.at[pl.ds(sid * CPT + h * HCPT, HCPT)], isrc, gsem)
        d2 = pltpu.async_copy(
            dstp.at[pl.ds(sid * CPT + h * HCPT, HCPT)], idst, gsem)
        d1.wait()
        d2.wait()

    def wgather_fire(j, c):
        pltpu.async_copy(ndp.at[idst.at[j]], wvals.at[j], wsem)
        return c

    def wgather_drain(j, c):
        pltpu.make_async_copy(ndp.at[idst.at[0]], wvals.at[0], wsem).wait()
        return c

    def wscatter_fire(j, c):
        pltpu.async_copy(wvals.at[j], wacc.at[isrc.at[j]], wsem, add=True)
        return c

    def half_phase(xref, h, do_w):
        load_half(h)
        if do_w:
            lax.fori_loop(0, HCPT, wgather_fire, 0)

        def grp(t, c):
            for b, (rows, ssem) in enumerate(((rows_a, sasem), (rows_b, sbsem))):
                j = 2 * t + b

                @pl.when(t >= 1)
                def _():
                    pltpu.make_async_copy(rows, acc.at[idst.at[0]], ssem).wait()
                pltpu.async_copy(xref.at[isrc.at[j]], rows, gsem).wait()
                pltpu.async_copy(rows, acc.at[idst.at[j]], ssem, add=True)
            return c

        lax.fori_loop(0, HCPT // 2, grp, 0)
        if do_w:
            lax.fori_loop(0, HCPT, wgather_drain, 0)
            lax.fori_loop(0, HCPT, wscatter_fire, 0)
            lax.fori_loop(0, HCPT, wgather_drain, 0)
        pltpu.make_async_copy(rows_a, acc.at[idst.at[0]], sasem).wait()
        pltpu.make_async_copy(rows_b, acc.at[idst.at[0]], sbsem).wait()

    @pl.when(cid == 0)
    def _():
        half_phase(x0, 0, True)
        half_phase(x0, 1, False)

    @pl.when(cid == 1)
    def _():
        half_phase(x1, 0, False)
        half_phase(x1, 1, True)

    plsc.subcore_barrier()

    @pl.when(cid == 0)
    def _():
        pltpu.sync_copy(acc.at[sl2], agg0.at[sl2])
        pltpu.sync_copy(wacc.at[sl1], w0.at[sl1])

    @pl.when(cid == 1)
    def _():
        pltpu.sync_copy(acc.at[sl2], agg1.at[sl2])
        pltpu.sync_copy(wacc.at[sl1], w1.at[sl1])


@functools.cache
def _sc_scatter():
    return pl.kernel(
        _sc_scatter_body,
        out_type=[
            jax.ShapeDtypeStruct((NPAD2, 128), jnp.float32),
            jax.ShapeDtypeStruct((NPAD2, 128), jnp.float32),
            jax.ShapeDtypeStruct((NPAD1,), jnp.float32),
            jax.ShapeDtypeStruct((NPAD1,), jnp.float32),
        ],
        mesh=_mesh(),
        scratch_types=[
            pltpu.VMEM((HCPT, CHUNK), jnp.int32),
            pltpu.VMEM((HCPT, CHUNK), jnp.int32),
            pltpu.VMEM((CHUNK, 128), jnp.float32),
            pltpu.VMEM((CHUNK, 128), jnp.float32),
            pltpu.VMEM((HCPT, CHUNK), jnp.float32),
            pltpu.VMEM_SHARED((NPAD2, 128), jnp.float32),
            pltpu.VMEM_SHARED((NPAD1,), jnp.float32),
            pltpu.SemaphoreType.DMA,
            pltpu.SemaphoreType.DMA,
            pltpu.SemaphoreType.DMA,
            pltpu.SemaphoreType.DMA,
        ],
    )


def _tc_prep_body(dof, dif, feat, x0_out, x1_out, ns_out, nd_out):
    do = dof[...]
    di = dif[...]
    ns = jnp.where(do > 0.0, lax.rsqrt(jnp.maximum(do, 1.0)), 0.0)
    nd = jnp.where(di > 0.0, lax.rsqrt(jnp.maximum(di, 1.0)), 0.0)
    ns_out[...] = ns
    nd_out[...] = nd
    f = feat[...]
    x0_out[...] = f[:, :128] * ns
    x1_out[...] = f[:, 128:] * ns


_PREP_BLK = 1000


def _tc_prep(deg_o, deg_i, in_feat):
    grid = N // _PREP_BLK
    xspec = pl.BlockSpec((_PREP_BLK, 128), lambda i: (i, 0))
    xshape = jax.ShapeDtypeStruct((N, 128), jnp.float32)
    nspec = pl.BlockSpec((_PREP_BLK, 1), lambda i: (i, 0))
    return pl.pallas_call(
        _tc_prep_body,
        grid=(grid,),
        in_specs=[
            nspec,
            nspec,
            pl.BlockSpec((_PREP_BLK, D_IN), lambda i: (i, 0)),
        ],
        out_specs=[xspec, xspec, nspec, nspec],
        out_shape=[xshape, xshape,
                   jax.ShapeDtypeStruct((N, 1), jnp.float32),
                   jax.ShapeDtypeStruct((N, 1), jnp.float32)],
    )(deg_o, deg_i, in_feat)


def _tc_final_body(a0, a1, ndr, nsr, u0, u1,
                   wa, wb, b1r, w2r, b2r, out, pooled):
    i = pl.program_id(0)
    nd = ndr[...]
    h = jnp.dot(a0[...] * nd, wa[...], preferred_element_type=jnp.float32)
    h = h + jnp.dot(a1[...] * nd, wb[...], preferred_element_type=jnp.float32)
    h = jnp.maximum(h + b1r[...], 0.0)
    coef = (u0[...] + u1[...]) * nsr[...]
    part = jnp.sum(h * coef, axis=0, keepdims=True)

    @pl.when(i == 0)
    def _():
        pooled[...] = part

    @pl.when(i > 0)
    def _():
        pooled[...] += part

    @pl.when(i == pl.num_programs(0) - 1)
    def _():
        out[...] = (jnp.dot(pooled[...] * (1.0 / N), w2r[...],
                            preferred_element_type=jnp.float32) + b2r[...])


def _tc_final(a0, a1, nd2, ns2, u0, u1, wa, wb, b1r, w2r, b2r):
    grid = N // _PREP_BLK
    aspec = pl.BlockSpec((_PREP_BLK, 128), lambda i: (i, 0))
    nspec = pl.BlockSpec((_PREP_BLK, 1), lambda i: (i, 0))
    wspec = pl.BlockSpec((128, D_H), lambda i: (0, 0))
    return pl.pallas_call(
        _tc_final_body,
        grid=(grid,),
        in_specs=[
            aspec, aspec,
            nspec, nspec, nspec, nspec,
            wspec, wspec,
            pl.BlockSpec((1, D_H), lambda i: (0, 0)),
            pl.BlockSpec((D_H, D_OUT), lambda i: (0, 0)),
            pl.BlockSpec((1, D_OUT), lambda i: (0, 0)),
        ],
        out_specs=pl.BlockSpec((1, D_OUT), lambda i: (0, 0)),
        out_shape=jax.ShapeDtypeStruct((1, D_OUT), jnp.float32),
        scratch_shapes=[pltpu.VMEM((1, D_H), jnp.float32)],
    )(a0, a1, nd2, ns2, u0, u1, wa, wb, b1r, w2r, b2r)


def kernel(in_feat, edge_index, W1, b1, W2, b2):
    src = edge_index[0]
    dst = edge_index[1]
    pad = jnp.full((EPAD - E,), N, jnp.int32)
    srcp = jnp.concatenate([src, pad]).reshape(NCH, CHUNK)
    dstp = jnp.concatenate([dst, pad]).reshape(NCH, CHUNK)
    ones_h = jnp.ones((CHUNK,), jnp.float32)
    z1 = jnp.zeros((RPT1,), jnp.float32)
    z2 = jnp.zeros((RPT2, 128), jnp.float32)

    deg_o, deg_i = _sc_degrees()(srcp, dstp, ones_h, z1)
    x0r, x1r, ns2, nd2 = _tc_prep(deg_o[:N, None], deg_i[:N, None], in_feat)
    xpad = ((0, NXP - N), (0, 0))
    ndp = jnp.pad(nd2[:, 0], (0, NXP - N))
    agg0, agg1, w0, w1 = _sc_scatter()(
        srcp, dstp, jnp.pad(x0r, xpad), jnp.pad(x1r, xpad), ndp, z2, z1)
    out = _tc_final(agg0[:N], agg1[:N], nd2, ns2,
                    w0[:N, None], w1[:N, None],
                    W1[:128], W1[128:], b1[None], W2, b2[None])
    return out.reshape(D_OUT)
